# baseline (speedup 1.0000x reference)
import jax
import jax.numpy as jnp
from jax import lax
from jax.experimental import pallas as pl
from jax.experimental.pallas import tpu as pltpu
from jax.experimental.pallas import tpu_sc as plsc

VOCAB = 1000000
EMB_DIM = 32
BATCH = 16384
FIELDS = 26
TOTAL = BATCH * FIELDS

_info = plsc.get_sparse_core_info()
NC, NS = _info.num_cores, _info.num_subcores
NW = NC * NS

CHUNK = 416
PER_W = TOTAL // NW
NCHUNK = PER_W // CHUNK
NBUF = 8
K = 4

assert PER_W % CHUNK == 0
assert NCHUNK % NBUF == 0
assert CHUNK % 8 == 0
assert NCHUNK >= NBUF


def _body(ids_hbm, table_hbm, out_hbm, idx_v, rows_v, *sems):
    gsem, osem = sems[:NBUF], sems[NBUF:]
    wid = lax.axis_index("s") * NC + lax.axis_index("c")
    chunk0 = wid * NCHUNK

    pltpu.sync_copy(ids_hbm.at[pl.ds(chunk0, NCHUNK)], idx_v)

    def start_gather(c, b):
        pltpu.async_copy(table_hbm.at[idx_v.at[c]], rows_v.at[b], gsem[b])

    def wait_gather(c, b):
        pltpu.make_async_copy(table_hbm.at[idx_v.at[c]], rows_v.at[b],
                              gsem[b]).wait()

    def start_scatter(c, b):
        pltpu.async_copy(rows_v.at[b],
                         out_hbm.at[pl.ds((chunk0 + c) * CHUNK, CHUNK)],
                         osem[b])

    def wait_scatter(c, b):
        pltpu.make_async_copy(rows_v.at[b],
                              out_hbm.at[pl.ds((chunk0 + c) * CHUNK, CHUNK)],
                              osem[b]).wait()

    for b in range(K):
        start_gather(b, b)

    def group(g):
        for b in range(NBUF):
            c = g * NBUF + b
            wait_gather(c, b)
            start_scatter(c, b)
            nb = (b + K) % NBUF

            @pl.when(c - (NBUF - K) >= 0)
            def _():
                wait_scatter(c - (NBUF - K), nb)

            @pl.when(c + K <= NCHUNK - 1)
            def _():
                start_gather(c + K, nb)

    pl.loop(0, NCHUNK // NBUF)(group)

    for i in range(NBUF - K):
        c = NCHUNK - (NBUF - K) + i
        wait_scatter(c, c % NBUF)


def kernel(ids, table):
    flat_ids = ids.reshape(-1).astype(jnp.int32)
    ids2d = flat_ids.reshape(TOTAL // CHUNK, CHUNK)

    mesh = plsc.VectorSubcoreMesh(core_axis_name="c", subcore_axis_name="s")
    out = pl.kernel(
        _body,
        out_type=jax.ShapeDtypeStruct((TOTAL, EMB_DIM), jnp.float32),
        mesh=mesh,
        scratch_types=(
            [pltpu.VMEM((NCHUNK, CHUNK), jnp.int32),
             pltpu.VMEM((NBUF, CHUNK, EMB_DIM), jnp.float32)]
            + [pltpu.SemaphoreType.DMA] * (2 * NBUF)
        ),
        compiler_params=pltpu.CompilerParams(use_tc_tiling_on_sc=False),
    )(ids2d, table)
    return out.reshape(BATCH, FIELDS, EMB_DIM)
